# Initial kernel scaffold; baseline (speedup 1.0000x reference)
import jax
import jax.numpy as jnp
from jax.experimental import pallas as pl


def kernel(ft_user, ft_item, bn_g_u, bn_b_u, bn_g_i, bn_b_i, Wq_ui, bq_ui, Wk_ui, Wv_ui, attn_ui, emb_cnt, Wq_ii, bq_ii, Wk_ii, Wv_ii, attn_ii, W_agg, b_agg, W_self, src_ui, dst_ui, src_ii, dst_ii, cnt_ui):
    raise NotImplementedError("write your pallas kernel here")


# speedup vs baseline: 4.1966x; 4.1966x over previous
import functools

import jax
import jax.numpy as jnp
from jax import lax
from jax.experimental import pallas as pl
from jax.experimental.pallas import tpu as pltpu
from jax.experimental.pallas import tpu_sc as plsc

_N = 10000
_D = 128
_E = 160000
_CH = 64
_NCHUNK = _E // _CH
_DC = _D // 16
_NW = 32
_CHUNKS_EVEN = _NCHUNK // _NW
_CHUNKS_REM = _NCHUNK - _NW * _CHUNKS_EVEN
_RB = 1000
_ROWS_PER_TILE = 624
_TAIL_ROWS = _N - 16 * _ROWS_PER_TILE
_ZCH = 48
_NZ = _ROWS_PER_TILE // _ZCH
_NP = 10240
_DR = _NP // 16


def _stats_body(u_ref, i_ref, o_ref):
    pid = pl.program_id(0)
    u = u_ref[...]
    it = i_ref[...]
    z = jnp.zeros((1, _D), jnp.float32)
    blk = jnp.concatenate(
        [
            jnp.sum(u, axis=0, keepdims=True),
            jnp.sum(u * u, axis=0, keepdims=True),
            jnp.sum(it, axis=0, keepdims=True),
            jnp.sum(it * it, axis=0, keepdims=True),
            z, z, z, z,
        ],
        axis=0,
    )

    @pl.when(pid == 0)
    def _():
        o_ref[...] = blk

    @pl.when(pid != 0)
    def _():
        o_ref[...] = o_ref[...] + blk


def _stats(ft_user, ft_item):
    grid = (_N // _RB,)
    return pl.pallas_call(
        _stats_body,
        grid=grid,
        in_specs=[
            pl.BlockSpec((_RB, _D), lambda i: (i, 0)),
            pl.BlockSpec((_RB, _D), lambda i: (i, 0)),
        ],
        out_specs=pl.BlockSpec((8, _D), lambda i: (0, 0)),
        out_shape=jax.ShapeDtypeStruct((8, _D), jnp.float32),
    )(ft_user, ft_item)


def _proj_body(u_ref, i_ref, st_ref, gu_ref, bu_ref, gi_ref, bi_ref,
               wq1_ref, bq1_ref, wk1_ref, wv1_ref,
               wq2_ref, bq2_ref, wk2_ref, wv2_ref,
               q1_ref, v1_ref, k1_ref, q2_ref, v2_ref, k2_ref, fti_ref):
    n = jnp.float32(_N)
    st = st_ref[...]
    mu_u = st[0:1] / n
    var_u = st[1:2] / n - mu_u * mu_u
    mu_i = st[2:3] / n
    var_i = st[3:4] / n - mu_i * mu_i
    rs_u = lax.rsqrt(var_u + 1e-5)
    rs_i = lax.rsqrt(var_i + 1e-5)
    ftu = (u_ref[...] - mu_u) * rs_u * gu_ref[...] + bu_ref[...]
    fti = (i_ref[...] - mu_i) * rs_i * gi_ref[...] + bi_ref[...]

    def mm(x, w):
        return jax.lax.dot_general(
            x, w, (((1,), (0,)), ((), ())),
            preferred_element_type=jnp.float32)

    q1_ref[...] = mm(ftu, wq1_ref[...]) + bq1_ref[...]
    v1_ref[...] = mm(ftu, wv1_ref[...])
    k1_ref[...] = mm(fti, wk1_ref[...])
    q2_ref[...] = mm(fti, wq2_ref[...]) + bq2_ref[...]
    v2_ref[...] = mm(fti, wv2_ref[...])
    k2_ref[...] = mm(fti, wk2_ref[...])
    fti_ref[...] = fti


def _proj(ft_user, ft_item, stats, bn_g_u, bn_b_u, bn_g_i, bn_b_i,
          Wq_ui, bq_ui, Wk_ui, Wv_ui, Wq_ii, bq_ii, Wk_ii, Wv_ii):
    grid = (_N // _RB,)
    row = lambda i: (i, 0)
    full = lambda i: (0, 0)
    vec = pl.BlockSpec((1, _D), full)
    mat = pl.BlockSpec((_D, _D), full)
    return pl.pallas_call(
        _proj_body,
        grid=grid,
        in_specs=[
            pl.BlockSpec((_RB, _D), row),
            pl.BlockSpec((_RB, _D), row),
            pl.BlockSpec((8, _D), full),
            vec, vec, vec, vec,
            mat, vec, mat, mat,
            mat, vec, mat, mat,
        ],
        out_specs=[pl.BlockSpec((_RB, _D), row)] * 7,
        out_shape=[jax.ShapeDtypeStruct((_N, _D), jnp.float32)] * 7,
    )(ft_user, ft_item, stats, bn_g_u, bn_b_u, bn_g_i, bn_b_i,
      Wq_ui, bq_ui, Wk_ui, Wv_ui, Wq_ii, bq_ii, Wk_ii, Wv_ii)


def _sc_body(q1, k1, v1, emb, attn1, src1, dst1, cnt1,
             q2, k2, v2, attn2, src2, dst2,
             agg_out, den_out,
             src_v, dst_v, cnt_v, dni_v, qb, kb, eb, pb, sbuf, abuf,
             agg_sh, den_sh):
    c = lax.axis_index("c")
    s = lax.axis_index("s")
    wid = c * 16 + s
    lane = lax.iota(jnp.int32, 16)
    zv = jnp.zeros((16,), jnp.float32)

    def zrow(i, car):
        for dc in range(_DC):
            qb[i, pl.ds(dc * 16, 16)] = zv
        return car
    lax.fori_loop(0, _CH, zrow, 0)

    def fill_idx(r0):
        for k in range(4):
            src_v[0, pl.ds(k * 16, 16)] = r0 + k * 16 + lane

    nzc = jnp.where(s == 15, 11, 10)

    def zcopy(i, car):
        r0 = jnp.where(i < 10,
                       s * _ROWS_PER_TILE + jnp.minimum(i * 64, 560),
                       _N - 64)
        fill_idx(r0)
        pltpu.sync_copy(qb, agg_sh.at[src_v.at[0]])
        return car
    lax.fori_loop(0, nzc, zcopy, 0)

    dbase = jnp.minimum(s * 64, _DR - 64)
    fill_idx(dbase)
    pltpu.sync_copy(qb, den_sh.at[src_v.at[0]])

    plsc.subcore_barrier()

    def load_attn(attn_hbm):
        pltpu.sync_copy(attn_hbm, abuf)
        avecs = [abuf[pl.ds(dc * 16, 16)] for dc in range(_DC)]
        m = zv
        for dc in range(_DC):
            m = m + jnp.maximum(avecs[dc], 0.0)
        return avecs, jnp.sum(m)

    def run_etype(q_tab, k_tab, v_tab, emb_tab, src_h, dst_h, cnt_h,
                  avecs, mshift):
        nch = jnp.where(wid < _CHUNKS_REM, _CHUNKS_EVEN + 1, _CHUNKS_EVEN)

        def chunk_body(j, car):
            base = (wid + _NW * j) * _CH
            pltpu.sync_copy(src_h.at[pl.ds(base, _CH)], src_v.at[0])
            pltpu.sync_copy(dst_h.at[pl.ds(base, _CH)], dst_v.at[0])
            if cnt_h is not None:
                pltpu.sync_copy(cnt_h.at[pl.ds(base, _CH)], cnt_v.at[0])
            pltpu.sync_copy(q_tab.at[src_v.at[0]], qb)
            pltpu.sync_copy(k_tab.at[dst_v.at[0]], kb)
            if emb_tab is not None:
                pltpu.sync_copy(emb_tab.at[cnt_v.at[0]], eb)

            def score_body(g, car2):
                svec = zv
                for l in range(16):
                    row = g * 16 + l
                    acc = zv
                    for dc in range(_DC):
                        x = (qb[row, pl.ds(dc * 16, 16)]
                             + kb[row, pl.ds(dc * 16, 16)])
                        if emb_tab is not None:
                            x = x + eb[row, pl.ds(dc * 16, 16)]
                        sg = 1.0 / (1.0 + jnp.exp(-x))
                        acc = acc + avecs[dc] * sg
                    svec = jnp.where(lane == l, jnp.sum(acc), svec)
                pv = jnp.exp(svec - mshift)
                sbuf[pl.ds(g * 16, 16)] = pv
                return car2

            lax.fori_loop(0, _CH // 16, score_body, 0)
            pltpu.sync_copy(v_tab.at[src_v.at[0]], eb)

            def scale_body(g, car2):
                pv = sbuf[pl.ds(g * 16, 16)]
                dvec = dst_v[0, pl.ds(g * 16, 16)]
                dni_v[0, pl.ds(g * 16, 16)] = dvec // 16
                dmv = dvec % 16
                for l in range(16):
                    row = g * 16 + l
                    p = pv[l]
                    dm = dmv[l]
                    ch8 = dm // 2
                    lpos = (dm % 2) * 8
                    for dc in range(_DC):
                        eb[row, pl.ds(dc * 16, 16)] = (
                            eb[row, pl.ds(dc * 16, 16)] * p)
                        pb[row, pl.ds(dc * 16, 16)] = jnp.where(
                            jnp.logical_and(ch8 == dc, lane == lpos), p, 0.0)
                return car2

            lax.fori_loop(0, _CH // 16, scale_body, 0)
            pltpu.sync_copy(eb, agg_sh.at[dst_v.at[0]], add=True)
            pltpu.sync_copy(pb, den_sh.at[dni_v.at[0]], add=True)
            return car

        lax.fori_loop(0, nch, chunk_body, 0)

    avecs1, m1 = load_attn(attn1)
    avecs2, m2 = load_attn(attn2)
    mshift = jnp.maximum(m1, m2)
    run_etype(q1, k1, v1, emb, src1, dst1, cnt1, avecs1, mshift)
    run_etype(q2, k2, v2, None, src2, dst2, None, avecs2, mshift)

    plsc.subcore_barrier()

    def dcopy(i, car):
        r0 = jnp.where(i < 10,
                       s * _ROWS_PER_TILE + jnp.minimum(i * 64, 560),
                       _N - 64)
        fill_idx(r0)
        pltpu.sync_copy(agg_sh.at[src_v.at[0]], qb)
        pltpu.sync_copy(qb, agg_out.at[c, pl.ds(r0, 64)])
        return car
    lax.fori_loop(0, nzc, dcopy, 0)

    fill_idx(dbase)
    pltpu.sync_copy(den_sh.at[src_v.at[0]], pb)
    pltpu.sync_copy(pb, den_out.at[c, pl.ds(dbase, 64)])


def _sc_edges(q1, k1, v1, emb, attn1, src1, dst1, cnt1,
              q2, k2, v2, attn2, src2, dst2):
    mesh = plsc.VectorSubcoreMesh(core_axis_name="c", subcore_axis_name="s")
    fn = pl.kernel(
        _sc_body,
        out_type=[
            jax.ShapeDtypeStruct((2, _N, _D), jnp.float32),
            jax.ShapeDtypeStruct((2, _DR, _D), jnp.float32),
        ],
        mesh=mesh,
        compiler_params=pltpu.CompilerParams(needs_layout_passes=False),
        scratch_types=[
            pltpu.VMEM((1, _CH), jnp.int32),
            pltpu.VMEM((1, _CH), jnp.int32),
            pltpu.VMEM((1, _CH), jnp.int32),
            pltpu.VMEM((1, _CH), jnp.int32),
            pltpu.VMEM((_CH, _D), jnp.float32),
            pltpu.VMEM((_CH, _D), jnp.float32),
            pltpu.VMEM((_CH, _D), jnp.float32),
            pltpu.VMEM((_CH, _D), jnp.float32),
            pltpu.VMEM((_CH,), jnp.float32),
            pltpu.VMEM((_D,), jnp.float32),
            pltpu.VMEM_SHARED((_N, _D), jnp.float32),
            pltpu.VMEM_SHARED((_DR, _D), jnp.float32),
        ],
    )
    return fn(q1, k1, v1, emb, attn1, src1, dst1, cnt1,
              q2, k2, v2, attn2, src2, dst2)


def _post_body(agg_ref, den_ref, fti_ref, wa_ref, ba_ref, ws_ref, o_ref):
    a = agg_ref[0] + agg_ref[1]
    d = (den_ref[:, 0] + den_ref[:, 1]).reshape(-1, 1)
    safe = jnp.where(d > 0.0, d, 1.0)
    w = a / safe

    def mm(x, wt):
        return jax.lax.dot_general(
            x, wt, (((1,), (0,)), ((), ())),
            preferred_element_type=jnp.float32)

    o_ref[...] = jnp.maximum(
        mm(w, wa_ref[...]) + ba_ref[...] + mm(fti_ref[...], ws_ref[...]), 0.0)


def _post(aggp, denp, fti_n, W_agg, b_agg, W_self):
    grid = (_N // _RB,)
    return pl.pallas_call(
        _post_body,
        grid=grid,
        in_specs=[
            pl.BlockSpec((2, _RB, _D), lambda i: (0, i, 0)),
            pl.BlockSpec((_RB, 2), lambda i: (i, 0)),
            pl.BlockSpec((_RB, _D), lambda i: (i, 0)),
            pl.BlockSpec((_D, _D), lambda i: (0, 0)),
            pl.BlockSpec((1, _D), lambda i: (0, 0)),
            pl.BlockSpec((_D, _D), lambda i: (0, 0)),
        ],
        out_specs=pl.BlockSpec((_RB, _D), lambda i: (i, 0)),
        out_shape=jax.ShapeDtypeStruct((_N, _D), jnp.float32),
    )(aggp, denp, fti_n, W_agg, b_agg, W_self)


def kernel(ft_user, ft_item, bn_g_u, bn_b_u, bn_g_i, bn_b_i,
           Wq_ui, bq_ui, Wk_ui, Wv_ui, attn_ui, emb_cnt,
           Wq_ii, bq_ii, Wk_ii, Wv_ii, attn_ii,
           W_agg, b_agg, W_self,
           src_ui, dst_ui, src_ii, dst_ii, cnt_ui):
    f32 = jnp.float32
    i32 = jnp.int32
    ft_user = ft_user.astype(f32)
    ft_item = ft_item.astype(f32)
    src_ui = src_ui.astype(i32)
    dst_ui = dst_ui.astype(i32)
    src_ii = src_ii.astype(i32)
    dst_ii = dst_ii.astype(i32)
    cnt_ui = cnt_ui.astype(i32)

    stats = _stats(ft_user, ft_item)
    q1, v1, k1, q2, v2, k2, fti_n = _proj(
        ft_user, ft_item, stats,
        bn_g_u.reshape(1, _D), bn_b_u.reshape(1, _D),
        bn_g_i.reshape(1, _D), bn_b_i.reshape(1, _D),
        Wq_ui, bq_ui.reshape(1, _D), Wk_ui, Wv_ui,
        Wq_ii, bq_ii.reshape(1, _D), Wk_ii, Wv_ii)
    aggp, denp = _sc_edges(
        q1, k1, v1, emb_cnt.astype(f32), attn_ui.astype(f32),
        src_ui, dst_ui, cnt_ui,
        q2, k2, v2, attn_ii.astype(f32), src_ii, dst_ii)
    den2 = denp.reshape(2, _DR, 16, 8)[:, :, :, 0].reshape(2, _NP)[:, :_N].T
    return _post(aggp, den2, fti_n, W_agg, b_agg.reshape(1, _D), W_self)
